# baseline (speedup 1.0000x reference)
import functools

import numpy as np

import jax
import jax.numpy as jnp
from jax import lax
from jax.experimental import pallas as pl
from jax.experimental.pallas import tpu as pltpu
from jax.experimental.pallas import tpu_sc as plsc

_NT = (((1,), (1,)), ((), ()))


def _make_sc_agg(n_nodes, n_edges, width, nc, ns, k, nbuf, n_phases, split,
                 dtype=jnp.float32, deg_width=0):
    edges_per_worker = n_edges // (ns if split else nc * ns)
    n_chunks = edges_per_worker // k
    assert edges_per_worker % k == 0 and n_chunks % n_phases == 0
    pchunks = n_chunks // n_phases
    assert pchunks % nbuf == 0
    rows_per_tile = n_nodes // ns
    mesh = plsc.VectorSubcoreMesh(core_axis_name="c", subcore_axis_name="s")

    out_type = [jax.ShapeDtypeStruct((nc, n_nodes, width), dtype)]
    scratch = [
        pltpu.VMEM((pchunks, k), jnp.int32),
        pltpu.VMEM((pchunks, k), jnp.int32),
        pltpu.VMEM((nbuf, k, width), dtype),
        pltpu.VMEM_SHARED((n_nodes, width), dtype),
        pltpu.SemaphoreType.DMA((nbuf,)),
        pltpu.SemaphoreType.DMA((nbuf,)),
    ]
    if deg_width:
        out_type.append(
            jax.ShapeDtypeStruct((nc, n_nodes, deg_width), jnp.float32))
        scratch += [
            pltpu.VMEM((k, deg_width), jnp.float32),
            pltpu.VMEM_SHARED((n_nodes, deg_width), jnp.float32),
            pltpu.SemaphoreType.DMA((nbuf,)),
        ]

    @functools.partial(pl.kernel, mesh=mesh, out_type=out_type,
                       compiler_params=pltpu.CompilerParams(
                           use_tc_tiling_on_sc=False),
                       scratch_types=scratch)
    def agg(*refs):
        if deg_width:
            (table_hbm, src_hbm, dst_hbm, zeros_hbm, zdeg_hbm, ones_hbm,
             out_hbm, deg_hbm, src_v, dst_v, rows_v, acc_sh, gsem, ssem,
             ones_v, deg_sh, dsem) = refs
        else:
            (table_hbm, src_hbm, dst_hbm, zeros_hbm, out_hbm,
             src_v, dst_v, rows_v, acc_sh, gsem, ssem) = refs
        c = lax.axis_index("c")
        s = lax.axis_index("s")
        tile_rows = pl.ds(s * rows_per_tile, rows_per_tile)
        pltpu.sync_copy(zeros_hbm, acc_sh.at[tile_rows])
        if deg_width:
            pltpu.sync_copy(zdeg_hbm, deg_sh.at[tile_rows])
            pltpu.sync_copy(ones_hbm, ones_v)
        plsc.subcore_barrier()

        for phase in range(n_phases):
            if split:
                base = s * n_chunks + phase * pchunks
                pltpu.sync_copy(src_hbm.at[c, pl.ds(base, pchunks)], src_v)
            else:
                base = (s * nc + c) * n_chunks + phase * pchunks
                pltpu.sync_copy(src_hbm.at[pl.ds(base, pchunks)], src_v)
            pltpu.sync_copy(dst_hbm.at[pl.ds(base, pchunks)], dst_v)

            for b in range(nbuf):
                pltpu.async_copy(table_hbm.at[src_v.at[b]], rows_v.at[b],
                                 gsem.at[b])

            def body(g, carry):
                j0 = g * nbuf
                for b in range(nbuf):
                    j = j0 + b
                    pltpu.make_async_copy(
                        table_hbm.at[src_v.at[j]], rows_v.at[b], gsem.at[b]).wait()
                    pltpu.async_copy(
                        rows_v.at[b], acc_sh.at[dst_v.at[j]], ssem.at[b], add=True)
                    if deg_width:
                        pltpu.async_copy(
                            ones_v, deg_sh.at[dst_v.at[j]], dsem.at[b], add=True)
                for b in range(nbuf):
                    j2 = j0 + nbuf + b

                    @pl.when(j2 < pchunks)
                    def _():
                        pltpu.make_async_copy(
                            rows_v.at[b], acc_sh.at[dst_v.at[j0 + b]],
                            ssem.at[b]).wait()
                        if deg_width:
                            pltpu.make_async_copy(
                                ones_v, deg_sh.at[dst_v.at[j0 + b]],
                                dsem.at[b]).wait()
                        pltpu.async_copy(
                            table_hbm.at[src_v.at[j2]], rows_v.at[b], gsem.at[b])

                return carry

            lax.fori_loop(0, pchunks // nbuf, body, 0)
            for b in range(nbuf):
                pltpu.make_async_copy(
                    rows_v.at[b], acc_sh.at[dst_v.at[pchunks - nbuf + b]],
                    ssem.at[b]).wait()
                if deg_width:
                    pltpu.make_async_copy(
                        ones_v, deg_sh.at[dst_v.at[pchunks - nbuf + b]],
                        dsem.at[b]).wait()

        plsc.subcore_barrier()
        pltpu.sync_copy(acc_sh.at[tile_rows], out_hbm.at[c, tile_rows])
        if deg_width:
            pltpu.sync_copy(deg_sh.at[tile_rows], deg_hbm.at[c, tile_rows])

    return agg


def _dense_mid(x, xagg, dego, ws1, wn1, b1, ws2, wn2, bn):
    n, d = x.shape
    h = ws1.shape[0]
    cdim = ws2.shape[0]
    hw = 2 * xagg.shape[-1]
    dw = dego.shape[-1]
    grid = n // bn

    def body(x_ref, agg_ref, deg_ref, ws1_ref, wn1_ref, b1_ref, ws2_ref,
             wn2_ref, h1_ref, h1r_ref, z2a_ref, hs2_ref):
        def unpack(a):
            u = lax.bitcast_convert_type(a, jnp.uint32)
            v0 = lax.bitcast_convert_type(u << 16, jnp.float32)
            v1 = lax.bitcast_convert_type(u & jnp.uint32(0xFFFF0000),
                                          jnp.float32)
            return jnp.concatenate([v0, v1], axis=1)
        xagg_b = jnp.concatenate(
            [unpack(agg_ref[0]), unpack(agg_ref[1])], axis=1)
        deg = jnp.maximum(deg_ref[0, :, 0:1], 1.0)
        mean = xagg_b / deg
        h1 = (lax.dot_general(x_ref[...], ws1_ref[...], _NT,
                              preferred_element_type=jnp.float32)
              + lax.dot_general(mean, wn1_ref[...], _NT,
                                preferred_element_type=jnp.float32)
              + b1_ref[...])
        h1_ref[...] = h1
        h1r = jnp.maximum(h1, 0.0)
        h1r_ref[...] = h1r
        z2 = lax.dot_general(h1r, wn2_ref[...], _NT,
                             preferred_element_type=jnp.float32)
        hs2_ref[...] = lax.dot_general(h1r, ws2_ref[...], _NT,
                                       preferred_element_type=jnp.float32)
        z2a_ref[...] = jnp.concatenate(
            [z2, jnp.ones((bn, cdim), jnp.float32)], axis=1)

    return pl.pallas_call(
        body,
        grid=(grid,),
        in_specs=[
            pl.BlockSpec((bn, d), lambda i: (i, 0)),
            pl.BlockSpec((2, bn, hw // 2), lambda i: (0, i, 0)),
            pl.BlockSpec((2, bn, dw), lambda i: (0, i, 0)),
            pl.BlockSpec((h, d), lambda i: (0, 0)),
            pl.BlockSpec((h, d), lambda i: (0, 0)),
            pl.BlockSpec((1, h), lambda i: (0, 0)),
            pl.BlockSpec((cdim, h), lambda i: (0, 0)),
            pl.BlockSpec((cdim, h), lambda i: (0, 0)),
        ],
        out_specs=[
            pl.BlockSpec((bn, h), lambda i: (i, 0)),
            pl.BlockSpec((bn, h), lambda i: (i, 0)),
            pl.BlockSpec((bn, 2 * cdim), lambda i: (i, 0)),
            pl.BlockSpec((bn, cdim), lambda i: (i, 0)),
        ],
        out_shape=[
            jax.ShapeDtypeStruct((n, h), jnp.float32),
            jax.ShapeDtypeStruct((n, h), jnp.float32),
            jax.ShapeDtypeStruct((n, 2 * cdim), jnp.float32),
            jax.ShapeDtypeStruct((n, cdim), jnp.float32),
        ],
    )(x, xagg, dego, ws1, wn1, b1, ws2, wn2)


def _dense_final(hs2, agg2, b2, bn):
    n, cdim = hs2.shape
    wa = agg2.shape[-1]
    grid = n // bn

    def body(hs2_ref, agg_ref, b2_ref, h2_ref):
        a = agg_ref[0] + agg_ref[1]
        deg = jnp.maximum(a[:, cdim:cdim + 1], 1.0)
        h2_ref[...] = hs2_ref[...] + a[:, :cdim] / deg + b2_ref[...]

    return pl.pallas_call(
        body,
        grid=(grid,),
        in_specs=[
            pl.BlockSpec((bn, cdim), lambda i: (i, 0)),
            pl.BlockSpec((2, bn, wa), lambda i: (0, i, 0)),
            pl.BlockSpec((1, cdim), lambda i: (0, 0)),
        ],
        out_specs=pl.BlockSpec((bn, cdim), lambda i: (i, 0)),
        out_shape=jax.ShapeDtypeStruct((n, cdim), jnp.float32),
    )(hs2, agg2, b2)


def kernel(x, edge_index, W_self1, W_neigh1, b1, W_self2, W_neigh2, b2):
    n, d = x.shape
    e = edge_index.shape[1]
    h = W_self1.shape[0]
    cdim = W_self2.shape[0]
    info = plsc.get_sparse_core_info()
    nc, ns = info.num_cores, info.num_subcores

    hw = d // 2
    dw = 8
    tab = x.reshape(-1).astype(jnp.bfloat16).reshape(nc * n, hw)

    k1, k2 = 200, 200
    src = edge_index[0]
    srcA = jnp.stack([2 * src, 2 * src + 1]).reshape(nc, e // k1, k1)
    dstA = edge_index[1].reshape(e // k1, k1)
    agg_fn1 = _make_sc_agg(n, e, hw, nc, ns, k=k1, nbuf=5, n_phases=1,
                           split=True, dtype=jnp.bfloat16, deg_width=dw)
    xagg, dego = agg_fn1(
        tab, srcA, dstA,
        jnp.zeros((n // ns, hw), jnp.bfloat16),
        jnp.zeros((n // ns, dw), jnp.float32),
        jnp.ones((k1, dw), jnp.float32))

    xagg_p = lax.bitcast_convert_type(
        xagg.reshape(nc, n, hw // 2, 2), jnp.float32)
    ph = np.concatenate([np.arange(0, hw, 2), np.arange(1, hw, 2)])
    perm = np.concatenate([ph, hw + ph])
    h1, h1r, z2a, hs2 = _dense_mid(
        x, xagg_p, dego, W_self1, W_neigh1[:, perm], b1.reshape(1, h),
        W_self2, W_neigh2, bn=1000)

    w2 = 2 * cdim
    srcC = src.reshape(e // k2, k2)
    dstC = edge_index[1].reshape(e // k2, k2)
    agg_fn2 = _make_sc_agg(n, e, w2, nc, ns, k=k2, nbuf=5, n_phases=1,
                           split=False)
    (agg2,) = agg_fn2(z2a, srcC, dstC, jnp.zeros((n // ns, w2), jnp.float32))

    h2 = _dense_final(hs2, agg2, b2.reshape(1, cdim), bn=1000)
    return (h2, h1, h1r)


# speedup vs baseline: 1.0025x; 1.0025x over previous
import functools

import numpy as np

import jax
import jax.numpy as jnp
from jax import lax
from jax.experimental import pallas as pl
from jax.experimental.pallas import tpu as pltpu
from jax.experimental.pallas import tpu_sc as plsc

_NT = (((1,), (1,)), ((), ()))


def _make_sc_agg(n_nodes, n_edges, width, nc, ns, k, nbuf, n_phases, split,
                 dtype=jnp.float32, deg_width=0):
    edges_per_worker = n_edges // (ns if split else nc * ns)
    n_chunks = edges_per_worker // k
    assert edges_per_worker % k == 0 and n_chunks % n_phases == 0
    pchunks = n_chunks // n_phases
    assert pchunks % nbuf == 0
    rows_per_tile = n_nodes // ns
    mesh = plsc.VectorSubcoreMesh(core_axis_name="c", subcore_axis_name="s")

    out_type = [jax.ShapeDtypeStruct((nc, n_nodes, width), dtype)]
    scratch = [
        pltpu.VMEM((pchunks, k), jnp.int32),
        pltpu.VMEM((pchunks, k), jnp.int32),
        pltpu.VMEM((nbuf, k, width), dtype),
        pltpu.VMEM_SHARED((n_nodes, width), dtype),
        pltpu.SemaphoreType.DMA((nbuf,)),
        pltpu.SemaphoreType.DMA((nbuf,)),
    ]
    if deg_width:
        out_type.append(
            jax.ShapeDtypeStruct((nc, n_nodes, deg_width), jnp.float32))
        scratch += [
            pltpu.VMEM((k, deg_width), jnp.float32),
            pltpu.VMEM_SHARED((n_nodes, deg_width), jnp.float32),
            pltpu.SemaphoreType.DMA((nbuf,)),
        ]

    @functools.partial(pl.kernel, mesh=mesh, out_type=out_type,
                       compiler_params=pltpu.CompilerParams(
                           use_tc_tiling_on_sc=False),
                       scratch_types=scratch)
    def agg(*refs):
        if deg_width:
            (table_hbm, src_hbm, dst_hbm, zeros_hbm, zdeg_hbm, ones_hbm,
             out_hbm, deg_hbm, src_v, dst_v, rows_v, acc_sh, gsem, ssem,
             ones_v, deg_sh, dsem) = refs
        else:
            (table_hbm, src_hbm, dst_hbm, zeros_hbm, out_hbm,
             src_v, dst_v, rows_v, acc_sh, gsem, ssem) = refs
        c = lax.axis_index("c")
        s = lax.axis_index("s")
        tile_rows = pl.ds(s * rows_per_tile, rows_per_tile)
        pltpu.sync_copy(zeros_hbm, acc_sh.at[tile_rows])
        if deg_width:
            pltpu.sync_copy(zdeg_hbm, deg_sh.at[tile_rows])
            pltpu.sync_copy(ones_hbm, ones_v)
        plsc.subcore_barrier()

        for phase in range(n_phases):
            if split:
                base = s * n_chunks + phase * pchunks
                pltpu.sync_copy(src_hbm.at[c, pl.ds(base, pchunks)], src_v)
            else:
                base = (s * nc + c) * n_chunks + phase * pchunks
                pltpu.sync_copy(src_hbm.at[pl.ds(base, pchunks)], src_v)
            pltpu.sync_copy(dst_hbm.at[pl.ds(base, pchunks)], dst_v)

            for b in range(nbuf):
                pltpu.async_copy(table_hbm.at[src_v.at[b]], rows_v.at[b],
                                 gsem.at[b])

            def body(g, carry):
                j0 = g * nbuf
                for b in range(nbuf):
                    j = j0 + b
                    pltpu.make_async_copy(
                        table_hbm.at[src_v.at[j]], rows_v.at[b], gsem.at[b]).wait()
                    pltpu.async_copy(
                        rows_v.at[b], acc_sh.at[dst_v.at[j]], ssem.at[b], add=True)
                    if deg_width:
                        pltpu.async_copy(
                            ones_v, deg_sh.at[dst_v.at[j]], dsem.at[b], add=True)
                for b in range(nbuf):
                    j2 = j0 + nbuf + b

                    @pl.when(j2 < pchunks)
                    def _():
                        pltpu.make_async_copy(
                            rows_v.at[b], acc_sh.at[dst_v.at[j0 + b]],
                            ssem.at[b]).wait()
                        if deg_width:
                            pltpu.make_async_copy(
                                ones_v, deg_sh.at[dst_v.at[j0 + b]],
                                dsem.at[b]).wait()
                        pltpu.async_copy(
                            table_hbm.at[src_v.at[j2]], rows_v.at[b], gsem.at[b])

                return carry

            lax.fori_loop(0, pchunks // nbuf, body, 0)
            for b in range(nbuf):
                pltpu.make_async_copy(
                    rows_v.at[b], acc_sh.at[dst_v.at[pchunks - nbuf + b]],
                    ssem.at[b]).wait()
                if deg_width:
                    pltpu.make_async_copy(
                        ones_v, deg_sh.at[dst_v.at[pchunks - nbuf + b]],
                        dsem.at[b]).wait()

        plsc.subcore_barrier()
        pltpu.sync_copy(acc_sh.at[tile_rows], out_hbm.at[c, tile_rows])
        if deg_width:
            pltpu.sync_copy(deg_sh.at[tile_rows], deg_hbm.at[c, tile_rows])

    return agg


def _dense_mid(x, xagg, dego, ws1, wn1, b1, ws2, wn2, bn):
    n, d = x.shape
    h = ws1.shape[0]
    cdim = ws2.shape[0]
    hw = 2 * xagg.shape[-1]
    dw = dego.shape[-1]
    grid = n // bn

    def body(x_ref, agg_ref, deg_ref, ws1_ref, wn1_ref, b1_ref, ws2_ref,
             wn2_ref, h1_ref, h1r_ref, z2a_ref, hs2_ref):
        def unpack(a):
            u = lax.bitcast_convert_type(a, jnp.uint32)
            v0 = lax.bitcast_convert_type(u << 16, jnp.float32)
            v1 = lax.bitcast_convert_type(u & jnp.uint32(0xFFFF0000),
                                          jnp.float32)
            return jnp.concatenate([v0, v1], axis=1)
        xagg_b = jnp.concatenate(
            [unpack(agg_ref[0]), unpack(agg_ref[1])], axis=1)
        deg = jnp.maximum(deg_ref[0, :, 0:1], 1.0)
        mean = xagg_b / deg
        h1 = (lax.dot_general(x_ref[...], ws1_ref[...], _NT,
                              preferred_element_type=jnp.float32)
              + lax.dot_general(mean, wn1_ref[...], _NT,
                                preferred_element_type=jnp.float32)
              + b1_ref[...])
        h1_ref[...] = h1
        h1r = jnp.maximum(h1, 0.0)
        h1r_ref[...] = h1r
        z2 = lax.dot_general(h1r, wn2_ref[...], _NT,
                             preferred_element_type=jnp.float32)
        hs2_ref[...] = lax.dot_general(h1r, ws2_ref[...], _NT,
                                       preferred_element_type=jnp.float32)
        z2a_ref[...] = jnp.concatenate(
            [z2, jnp.ones((bn, cdim), jnp.float32)], axis=1)

    return pl.pallas_call(
        body,
        grid=(grid,),
        in_specs=[
            pl.BlockSpec((bn, d), lambda i: (i, 0)),
            pl.BlockSpec((2, bn, hw // 2), lambda i: (0, i, 0)),
            pl.BlockSpec((2, bn, dw), lambda i: (0, i, 0)),
            pl.BlockSpec((h, d), lambda i: (0, 0)),
            pl.BlockSpec((h, d), lambda i: (0, 0)),
            pl.BlockSpec((1, h), lambda i: (0, 0)),
            pl.BlockSpec((cdim, h), lambda i: (0, 0)),
            pl.BlockSpec((cdim, h), lambda i: (0, 0)),
        ],
        out_specs=[
            pl.BlockSpec((bn, h), lambda i: (i, 0)),
            pl.BlockSpec((bn, h), lambda i: (i, 0)),
            pl.BlockSpec((bn, 2 * cdim), lambda i: (i, 0)),
            pl.BlockSpec((bn, cdim), lambda i: (i, 0)),
        ],
        out_shape=[
            jax.ShapeDtypeStruct((n, h), jnp.float32),
            jax.ShapeDtypeStruct((n, h), jnp.float32),
            jax.ShapeDtypeStruct((n, 2 * cdim), jnp.float32),
            jax.ShapeDtypeStruct((n, cdim), jnp.float32),
        ],
    )(x, xagg, dego, ws1, wn1, b1, ws2, wn2)


def _dense_final(hs2, agg2, b2, bn):
    n, cdim = hs2.shape
    wa = agg2.shape[-1]
    grid = n // bn

    def body(hs2_ref, agg_ref, b2_ref, h2_ref):
        a = agg_ref[0] + agg_ref[1]
        deg = jnp.maximum(a[:, cdim:cdim + 1], 1.0)
        h2_ref[...] = hs2_ref[...] + a[:, :cdim] / deg + b2_ref[...]

    return pl.pallas_call(
        body,
        grid=(grid,),
        in_specs=[
            pl.BlockSpec((bn, cdim), lambda i: (i, 0)),
            pl.BlockSpec((2, bn, wa), lambda i: (0, i, 0)),
            pl.BlockSpec((1, cdim), lambda i: (0, 0)),
        ],
        out_specs=pl.BlockSpec((bn, cdim), lambda i: (i, 0)),
        out_shape=jax.ShapeDtypeStruct((n, cdim), jnp.float32),
    )(hs2, agg2, b2)


def kernel(x, edge_index, W_self1, W_neigh1, b1, W_self2, W_neigh2, b2):
    n, d = x.shape
    e = edge_index.shape[1]
    h = W_self1.shape[0]
    cdim = W_self2.shape[0]
    info = plsc.get_sparse_core_info()
    nc, ns = info.num_cores, info.num_subcores

    hw = d // 2
    dw = 8
    tab = x.astype(jnp.bfloat16).reshape(nc * n, hw)

    k1, k2 = 200, 200
    src = edge_index[0]
    srcA = jnp.stack([2 * src, 2 * src + 1]).reshape(nc, e // k1, k1)
    dstA = edge_index[1].reshape(e // k1, k1)
    agg_fn1 = _make_sc_agg(n, e, hw, nc, ns, k=k1, nbuf=5, n_phases=1,
                           split=True, dtype=jnp.bfloat16, deg_width=dw)
    xagg, dego = agg_fn1(
        tab, srcA, dstA,
        jnp.zeros((n // ns, hw), jnp.bfloat16),
        jnp.zeros((n // ns, dw), jnp.float32),
        jnp.ones((k1, dw), jnp.float32))

    xagg_p = lax.bitcast_convert_type(
        xagg.reshape(nc, n, hw // 2, 2), jnp.float32)
    ph = np.concatenate([np.arange(0, hw, 2), np.arange(1, hw, 2)])
    perm = np.concatenate([ph, hw + ph])
    h1, h1r, z2a, hs2 = _dense_mid(
        x, xagg_p, dego, W_self1, W_neigh1[:, perm], b1.reshape(1, h),
        W_self2, W_neigh2, bn=1000)

    w2 = 2 * cdim
    srcC = src.reshape(e // k2, k2)
    dstC = edge_index[1].reshape(e // k2, k2)
    agg_fn2 = _make_sc_agg(n, e, w2, nc, ns, k=k2, nbuf=5, n_phases=1,
                           split=False)
    (agg2,) = agg_fn2(z2a, srcC, dstC, jnp.zeros((n // ns, w2), jnp.float32))

    h2 = _dense_final(hs2, agg2, b2.reshape(1, cdim), bn=1000)
    return (h2, h1, h1r)


# speedup vs baseline: 1.1617x; 1.1588x over previous
import functools

import numpy as np

import jax
import jax.numpy as jnp
from jax import lax
from jax.experimental import pallas as pl
from jax.experimental.pallas import tpu as pltpu
from jax.experimental.pallas import tpu_sc as plsc

_NT = (((1,), (1,)), ((), ()))


def _make_sc_agg(n_nodes, n_edges, width, nc, ns, k, nbuf, n_phases, split,
                 dtype=jnp.float32, deg_width=0):
    edges_per_worker = n_edges // (ns if split else nc * ns)
    n_chunks = edges_per_worker // k
    assert edges_per_worker % k == 0 and n_chunks % n_phases == 0
    pchunks = n_chunks // n_phases
    assert pchunks % nbuf == 0
    rows_per_tile = n_nodes // ns
    mesh = plsc.VectorSubcoreMesh(core_axis_name="c", subcore_axis_name="s")

    out_type = [jax.ShapeDtypeStruct((nc, n_nodes, width), dtype)]
    scratch = [
        pltpu.VMEM((pchunks, k), jnp.int32),
        pltpu.VMEM((pchunks, k), jnp.int32),
        pltpu.VMEM((nbuf, k, width), dtype),
        pltpu.VMEM_SHARED((n_nodes, width), dtype),
        pltpu.SemaphoreType.DMA((nbuf,)),
        pltpu.SemaphoreType.DMA((nbuf,)),
    ]
    if deg_width:
        out_type.append(
            jax.ShapeDtypeStruct((nc, n_nodes, deg_width), jnp.float32))
        scratch += [
            pltpu.VMEM((k, deg_width), jnp.float32),
            pltpu.VMEM_SHARED((n_nodes, deg_width), jnp.float32),
            pltpu.SemaphoreType.DMA((nbuf,)),
        ]

    @functools.partial(pl.kernel, mesh=mesh, out_type=out_type,
                       compiler_params=pltpu.CompilerParams(
                           use_tc_tiling_on_sc=False),
                       scratch_types=scratch)
    def agg(*refs):
        if deg_width:
            (table_hbm, src_hbm, dst_hbm, zeros_hbm, zdeg_hbm, ones_hbm,
             out_hbm, deg_hbm, src_v, dst_v, rows_v, acc_sh, gsem, ssem,
             ones_v, deg_sh, dsem) = refs
        else:
            (table_hbm, src_hbm, dst_hbm, zeros_hbm, out_hbm,
             src_v, dst_v, rows_v, acc_sh, gsem, ssem) = refs
        c = lax.axis_index("c")
        s = lax.axis_index("s")
        tile_rows = pl.ds(s * rows_per_tile, rows_per_tile)
        pltpu.sync_copy(zeros_hbm, acc_sh.at[tile_rows])
        if deg_width:
            pltpu.sync_copy(zdeg_hbm, deg_sh.at[tile_rows])
            pltpu.sync_copy(ones_hbm, ones_v)
        plsc.subcore_barrier()

        for phase in range(n_phases):
            if split:
                base = s * n_chunks + phase * pchunks
                pltpu.sync_copy(src_hbm.at[c, pl.ds(base, pchunks)], src_v)
            else:
                base = (s * nc + c) * n_chunks + phase * pchunks
                pltpu.sync_copy(src_hbm.at[pl.ds(base, pchunks)], src_v)
            pltpu.sync_copy(dst_hbm.at[pl.ds(base, pchunks)], dst_v)

            for b in range(nbuf):
                pltpu.async_copy(table_hbm.at[src_v.at[b]], rows_v.at[b],
                                 gsem.at[b])

            def body(g, carry):
                j0 = g * nbuf
                for b in range(nbuf):
                    j = j0 + b
                    pltpu.make_async_copy(
                        table_hbm.at[src_v.at[j]], rows_v.at[b], gsem.at[b]).wait()
                    pltpu.async_copy(
                        rows_v.at[b], acc_sh.at[dst_v.at[j]], ssem.at[b], add=True)
                    if deg_width:
                        pltpu.async_copy(
                            ones_v, deg_sh.at[dst_v.at[j]], dsem.at[b], add=True)
                for b in range(nbuf):
                    j2 = j0 + nbuf + b

                    @pl.when(j2 < pchunks)
                    def _():
                        pltpu.make_async_copy(
                            rows_v.at[b], acc_sh.at[dst_v.at[j0 + b]],
                            ssem.at[b]).wait()
                        if deg_width:
                            pltpu.make_async_copy(
                                ones_v, deg_sh.at[dst_v.at[j0 + b]],
                                dsem.at[b]).wait()
                        pltpu.async_copy(
                            table_hbm.at[src_v.at[j2]], rows_v.at[b], gsem.at[b])

                return carry

            lax.fori_loop(0, pchunks // nbuf, body, 0)
            for b in range(nbuf):
                pltpu.make_async_copy(
                    rows_v.at[b], acc_sh.at[dst_v.at[pchunks - nbuf + b]],
                    ssem.at[b]).wait()
                if deg_width:
                    pltpu.make_async_copy(
                        ones_v, deg_sh.at[dst_v.at[pchunks - nbuf + b]],
                        dsem.at[b]).wait()

        plsc.subcore_barrier()
        pltpu.sync_copy(acc_sh.at[tile_rows], out_hbm.at[c, tile_rows])
        if deg_width:
            pltpu.sync_copy(deg_sh.at[tile_rows], deg_hbm.at[c, tile_rows])

    return agg


def _dense_mid(x, xagg, dego, ws1, wn1, b1, ws2, wn2, bn):
    n, d = x.shape
    h = ws1.shape[0]
    cdim = ws2.shape[0]
    hw = xagg.shape[-1]
    dw = dego.shape[-1]
    grid = n // bn

    def body(x_ref, agg_ref, deg_ref, ws1_ref, wn1_ref, b1_ref, ws2_ref,
             wn2_ref, h1_ref, h1r_ref, z2a_ref, hs2_ref):
        xagg_b = jnp.concatenate(
            [agg_ref[0].astype(jnp.float32), agg_ref[1].astype(jnp.float32)],
            axis=1)
        deg = jnp.maximum(deg_ref[0, :, 0:1], 1.0)
        mean = xagg_b / deg
        h1 = (lax.dot_general(x_ref[...], ws1_ref[...], _NT,
                              preferred_element_type=jnp.float32)
              + lax.dot_general(mean, wn1_ref[...], _NT,
                                preferred_element_type=jnp.float32)
              + b1_ref[...])
        h1_ref[...] = h1
        h1r = jnp.maximum(h1, 0.0)
        h1r_ref[...] = h1r
        z2 = lax.dot_general(h1r, wn2_ref[...], _NT,
                             preferred_element_type=jnp.float32)
        hs2_ref[...] = lax.dot_general(h1r, ws2_ref[...], _NT,
                                       preferred_element_type=jnp.float32)
        z2a_ref[...] = jnp.concatenate(
            [z2, jnp.ones((bn, cdim), jnp.float32)], axis=1)

    return pl.pallas_call(
        body,
        grid=(grid,),
        in_specs=[
            pl.BlockSpec((bn, d), lambda i: (i, 0)),
            pl.BlockSpec((2, bn, hw), lambda i: (0, i, 0)),
            pl.BlockSpec((2, bn, dw), lambda i: (0, i, 0)),
            pl.BlockSpec((h, d), lambda i: (0, 0)),
            pl.BlockSpec((h, d), lambda i: (0, 0)),
            pl.BlockSpec((1, h), lambda i: (0, 0)),
            pl.BlockSpec((cdim, h), lambda i: (0, 0)),
            pl.BlockSpec((cdim, h), lambda i: (0, 0)),
        ],
        out_specs=[
            pl.BlockSpec((bn, h), lambda i: (i, 0)),
            pl.BlockSpec((bn, h), lambda i: (i, 0)),
            pl.BlockSpec((bn, 2 * cdim), lambda i: (i, 0)),
            pl.BlockSpec((bn, cdim), lambda i: (i, 0)),
        ],
        out_shape=[
            jax.ShapeDtypeStruct((n, h), jnp.float32),
            jax.ShapeDtypeStruct((n, h), jnp.float32),
            jax.ShapeDtypeStruct((n, 2 * cdim), jnp.float32),
            jax.ShapeDtypeStruct((n, cdim), jnp.float32),
        ],
    )(x, xagg, dego, ws1, wn1, b1, ws2, wn2)


def _dense_final(hs2, agg2, b2, bn):
    n, cdim = hs2.shape
    wa = agg2.shape[-1]
    grid = n // bn

    def body(hs2_ref, agg_ref, b2_ref, h2_ref):
        a = agg_ref[0] + agg_ref[1]
        deg = jnp.maximum(a[:, cdim:cdim + 1], 1.0)
        h2_ref[...] = hs2_ref[...] + a[:, :cdim] / deg + b2_ref[...]

    return pl.pallas_call(
        body,
        grid=(grid,),
        in_specs=[
            pl.BlockSpec((bn, cdim), lambda i: (i, 0)),
            pl.BlockSpec((2, bn, wa), lambda i: (0, i, 0)),
            pl.BlockSpec((1, cdim), lambda i: (0, 0)),
        ],
        out_specs=pl.BlockSpec((bn, cdim), lambda i: (i, 0)),
        out_shape=jax.ShapeDtypeStruct((n, cdim), jnp.float32),
    )(hs2, agg2, b2)


def kernel(x, edge_index, W_self1, W_neigh1, b1, W_self2, W_neigh2, b2):
    n, d = x.shape
    e = edge_index.shape[1]
    h = W_self1.shape[0]
    cdim = W_self2.shape[0]
    info = plsc.get_sparse_core_info()
    nc, ns = info.num_cores, info.num_subcores

    hw = d // 2
    dw = 8
    tab = x.astype(jnp.bfloat16).reshape(nc * n, hw)

    k1, k2 = 200, 200
    src = edge_index[0]
    srcA = jnp.stack([2 * src, 2 * src + 1]).reshape(nc, e // k1, k1)
    dstA = edge_index[1].reshape(e // k1, k1)
    agg_fn1 = _make_sc_agg(n, e, hw, nc, ns, k=k1, nbuf=5, n_phases=1,
                           split=True, dtype=jnp.bfloat16, deg_width=dw)
    xagg, dego = agg_fn1(
        tab, srcA, dstA,
        jnp.zeros((n // ns, hw), jnp.bfloat16),
        jnp.zeros((n // ns, dw), jnp.float32),
        jnp.ones((k1, dw), jnp.float32))

    h1, h1r, z2a, hs2 = _dense_mid(
        x, xagg, dego, W_self1, W_neigh1, b1.reshape(1, h),
        W_self2, W_neigh2, bn=1000)

    w2 = 2 * cdim
    srcC = src.reshape(e // k2, k2)
    dstC = edge_index[1].reshape(e // k2, k2)
    agg_fn2 = _make_sc_agg(n, e, w2, nc, ns, k=k2, nbuf=5, n_phases=1,
                           split=False)
    (agg2,) = agg_fn2(z2a, srcC, dstC, jnp.zeros((n // ns, w2), jnp.float32))

    h2 = _dense_final(hs2, agg2, b2.reshape(1, cdim), bn=1000)
    return (h2, h1, h1r)


# speedup vs baseline: 1.1623x; 1.0005x over previous
import functools

import numpy as np

import jax
import jax.numpy as jnp
from jax import lax
from jax.experimental import pallas as pl
from jax.experimental.pallas import tpu as pltpu
from jax.experimental.pallas import tpu_sc as plsc

_NT = (((1,), (1,)), ((), ()))


def _make_sc_agg(n_nodes, n_edges, width, nc, ns, k, nbuf, n_phases, split,
                 dtype=jnp.float32, deg_width=0):
    edges_per_worker = n_edges // (ns if split else nc * ns)
    n_chunks = edges_per_worker // k
    assert edges_per_worker % k == 0 and n_chunks % n_phases == 0
    pchunks = n_chunks // n_phases
    assert pchunks % nbuf == 0
    rows_per_tile = n_nodes // ns
    mesh = plsc.VectorSubcoreMesh(core_axis_name="c", subcore_axis_name="s")

    out_type = [jax.ShapeDtypeStruct((nc, n_nodes, width), dtype)]
    scratch = [
        pltpu.VMEM((pchunks, k), jnp.int32),
        pltpu.VMEM((pchunks, k), jnp.int32),
        pltpu.VMEM((nbuf, k, width), dtype),
        pltpu.VMEM_SHARED((n_nodes, width), dtype),
        pltpu.SemaphoreType.DMA((nbuf,)),
        pltpu.SemaphoreType.DMA((nbuf,)),
    ]
    if deg_width:
        out_type.append(
            jax.ShapeDtypeStruct((nc, n_nodes, deg_width), jnp.float32))
        scratch += [
            pltpu.VMEM((k, deg_width), jnp.float32),
            pltpu.VMEM_SHARED((n_nodes, deg_width), jnp.float32),
            pltpu.SemaphoreType.DMA((nbuf,)),
        ]

    @functools.partial(pl.kernel, mesh=mesh, out_type=out_type,
                       compiler_params=pltpu.CompilerParams(
                           use_tc_tiling_on_sc=False,
                           skip_device_barrier=True),
                       scratch_types=scratch)
    def agg(*refs):
        if deg_width:
            (table_hbm, src_hbm, dst_hbm, zeros_hbm, zdeg_hbm, ones_hbm,
             out_hbm, deg_hbm, src_v, dst_v, rows_v, acc_sh, gsem, ssem,
             ones_v, deg_sh, dsem) = refs
        else:
            (table_hbm, src_hbm, dst_hbm, zeros_hbm, out_hbm,
             src_v, dst_v, rows_v, acc_sh, gsem, ssem) = refs
        c = lax.axis_index("c")
        s = lax.axis_index("s")
        tile_rows = pl.ds(s * rows_per_tile, rows_per_tile)
        pltpu.sync_copy(zeros_hbm, acc_sh.at[tile_rows])
        if deg_width:
            pltpu.sync_copy(zdeg_hbm, deg_sh.at[tile_rows])
            pltpu.sync_copy(ones_hbm, ones_v)
        plsc.subcore_barrier()

        for phase in range(n_phases):
            if split:
                base = s * n_chunks + phase * pchunks
                pltpu.sync_copy(src_hbm.at[c, pl.ds(base, pchunks)], src_v)
            else:
                base = (s * nc + c) * n_chunks + phase * pchunks
                pltpu.sync_copy(src_hbm.at[pl.ds(base, pchunks)], src_v)
            pltpu.sync_copy(dst_hbm.at[pl.ds(base, pchunks)], dst_v)

            for b in range(nbuf):
                pltpu.async_copy(table_hbm.at[src_v.at[b]], rows_v.at[b],
                                 gsem.at[b])

            def body(g, carry):
                j0 = g * nbuf
                for b in range(nbuf):
                    j = j0 + b
                    pltpu.make_async_copy(
                        table_hbm.at[src_v.at[j]], rows_v.at[b], gsem.at[b]).wait()
                    pltpu.async_copy(
                        rows_v.at[b], acc_sh.at[dst_v.at[j]], ssem.at[b], add=True)
                    if deg_width:
                        pltpu.async_copy(
                            ones_v, deg_sh.at[dst_v.at[j]], dsem.at[b], add=True)
                for b in range(nbuf):
                    j2 = j0 + nbuf + b

                    @pl.when(j2 < pchunks)
                    def _():
                        pltpu.make_async_copy(
                            rows_v.at[b], acc_sh.at[dst_v.at[j0 + b]],
                            ssem.at[b]).wait()
                        if deg_width:
                            pltpu.make_async_copy(
                                ones_v, deg_sh.at[dst_v.at[j0 + b]],
                                dsem.at[b]).wait()
                        pltpu.async_copy(
                            table_hbm.at[src_v.at[j2]], rows_v.at[b], gsem.at[b])

                return carry

            lax.fori_loop(0, pchunks // nbuf, body, 0)
            for b in range(nbuf):
                pltpu.make_async_copy(
                    rows_v.at[b], acc_sh.at[dst_v.at[pchunks - nbuf + b]],
                    ssem.at[b]).wait()
                if deg_width:
                    pltpu.make_async_copy(
                        ones_v, deg_sh.at[dst_v.at[pchunks - nbuf + b]],
                        dsem.at[b]).wait()

        plsc.subcore_barrier()
        pltpu.sync_copy(acc_sh.at[tile_rows], out_hbm.at[c, tile_rows])
        if deg_width:
            pltpu.sync_copy(deg_sh.at[tile_rows], deg_hbm.at[c, tile_rows])

    return agg


def _dense_mid(x, xagg, dego, ws1, wn1, b1, ws2, wn2, bn):
    n, d = x.shape
    h = ws1.shape[0]
    cdim = ws2.shape[0]
    hw = xagg.shape[-1]
    dw = dego.shape[-1]
    grid = n // bn

    def body(x_ref, agg_ref, deg_ref, ws1_ref, wn1_ref, b1_ref, ws2_ref,
             wn2_ref, h1_ref, h1r_ref, z2a_ref, hs2_ref):
        xagg_b = jnp.concatenate(
            [agg_ref[0].astype(jnp.float32), agg_ref[1].astype(jnp.float32)],
            axis=1)
        deg = jnp.maximum(deg_ref[0, :, 0:1], 1.0)
        mean = xagg_b / deg
        h1 = (lax.dot_general(x_ref[...], ws1_ref[...], _NT,
                              preferred_element_type=jnp.float32)
              + lax.dot_general(mean, wn1_ref[...], _NT,
                                preferred_element_type=jnp.float32)
              + b1_ref[...])
        h1_ref[...] = h1
        h1r = jnp.maximum(h1, 0.0)
        h1r_ref[...] = h1r
        z2 = lax.dot_general(h1r, wn2_ref[...], _NT,
                             preferred_element_type=jnp.float32)
        hs2_ref[...] = lax.dot_general(h1r, ws2_ref[...], _NT,
                                       preferred_element_type=jnp.float32)
        z2a_ref[...] = jnp.concatenate(
            [z2, jnp.ones((bn, cdim), jnp.float32)], axis=1)

    return pl.pallas_call(
        body,
        grid=(grid,),
        in_specs=[
            pl.BlockSpec((bn, d), lambda i: (i, 0)),
            pl.BlockSpec((2, bn, hw), lambda i: (0, i, 0)),
            pl.BlockSpec((2, bn, dw), lambda i: (0, i, 0)),
            pl.BlockSpec((h, d), lambda i: (0, 0)),
            pl.BlockSpec((h, d), lambda i: (0, 0)),
            pl.BlockSpec((1, h), lambda i: (0, 0)),
            pl.BlockSpec((cdim, h), lambda i: (0, 0)),
            pl.BlockSpec((cdim, h), lambda i: (0, 0)),
        ],
        out_specs=[
            pl.BlockSpec((bn, h), lambda i: (i, 0)),
            pl.BlockSpec((bn, h), lambda i: (i, 0)),
            pl.BlockSpec((bn, 2 * cdim), lambda i: (i, 0)),
            pl.BlockSpec((bn, cdim), lambda i: (i, 0)),
        ],
        out_shape=[
            jax.ShapeDtypeStruct((n, h), jnp.float32),
            jax.ShapeDtypeStruct((n, h), jnp.float32),
            jax.ShapeDtypeStruct((n, 2 * cdim), jnp.float32),
            jax.ShapeDtypeStruct((n, cdim), jnp.float32),
        ],
    )(x, xagg, dego, ws1, wn1, b1, ws2, wn2)


def _dense_final(hs2, agg2, b2, bn):
    n, cdim = hs2.shape
    wa = agg2.shape[-1]
    grid = n // bn

    def body(hs2_ref, agg_ref, b2_ref, h2_ref):
        a = agg_ref[0] + agg_ref[1]
        deg = jnp.maximum(a[:, cdim:cdim + 1], 1.0)
        h2_ref[...] = hs2_ref[...] + a[:, :cdim] / deg + b2_ref[...]

    return pl.pallas_call(
        body,
        grid=(grid,),
        in_specs=[
            pl.BlockSpec((bn, cdim), lambda i: (i, 0)),
            pl.BlockSpec((2, bn, wa), lambda i: (0, i, 0)),
            pl.BlockSpec((1, cdim), lambda i: (0, 0)),
        ],
        out_specs=pl.BlockSpec((bn, cdim), lambda i: (i, 0)),
        out_shape=jax.ShapeDtypeStruct((n, cdim), jnp.float32),
    )(hs2, agg2, b2)


def kernel(x, edge_index, W_self1, W_neigh1, b1, W_self2, W_neigh2, b2):
    n, d = x.shape
    e = edge_index.shape[1]
    h = W_self1.shape[0]
    cdim = W_self2.shape[0]
    info = plsc.get_sparse_core_info()
    nc, ns = info.num_cores, info.num_subcores

    hw = d // 2
    dw = 8
    tab = x.astype(jnp.bfloat16).reshape(nc * n, hw)

    k1, k2 = 200, 200
    src = edge_index[0]
    srcA = jnp.stack([2 * src, 2 * src + 1]).reshape(nc, e // k1, k1)
    dstA = edge_index[1].reshape(e // k1, k1)
    agg_fn1 = _make_sc_agg(n, e, hw, nc, ns, k=k1, nbuf=5, n_phases=1,
                           split=True, dtype=jnp.bfloat16, deg_width=dw)
    xagg, dego = agg_fn1(
        tab, srcA, dstA,
        jnp.zeros((n // ns, hw), jnp.bfloat16),
        jnp.zeros((n // ns, dw), jnp.float32),
        jnp.ones((k1, dw), jnp.float32))

    h1, h1r, z2a, hs2 = _dense_mid(
        x, xagg, dego, W_self1, W_neigh1, b1.reshape(1, h),
        W_self2, W_neigh2, bn=1000)

    w2 = 2 * cdim
    srcC = src.reshape(e // k2, k2)
    dstC = edge_index[1].reshape(e // k2, k2)
    agg_fn2 = _make_sc_agg(n, e, w2, nc, ns, k=k2, nbuf=5, n_phases=1,
                           split=False)
    (agg2,) = agg_fn2(z2a, srcC, dstC, jnp.zeros((n // ns, w2), jnp.float32))

    h2 = _dense_final(hs2, agg2, b2.reshape(1, cdim), bn=1000)
    return (h2, h1, h1r)


# speedup vs baseline: 1.2714x; 1.0939x over previous
import functools

import numpy as np

import jax
import jax.numpy as jnp
from jax import lax
from jax.experimental import pallas as pl
from jax.experimental.pallas import tpu as pltpu
from jax.experimental.pallas import tpu_sc as plsc

_NT = (((1,), (1,)), ((), ()))


def _make_sc_agg(n_nodes, n_edges, width, nc, ns, k, nbuf, n_phases, split,
                 dtype=jnp.float32, deg_width=0):
    edges_per_worker = n_edges // (ns if split else nc * ns)
    n_chunks = edges_per_worker // k
    assert edges_per_worker % k == 0 and n_chunks % n_phases == 0
    pchunks = n_chunks // n_phases
    assert pchunks % nbuf == 0
    rows_per_tile = n_nodes // ns
    mesh = plsc.VectorSubcoreMesh(core_axis_name="c", subcore_axis_name="s")

    out_type = [jax.ShapeDtypeStruct((nc, n_nodes, width), dtype)]
    scratch = [
        pltpu.VMEM((pchunks * k,) if split else (pchunks, k), jnp.int32),
        pltpu.VMEM((pchunks, k), jnp.int32),
        pltpu.VMEM((nbuf, k, width), dtype),
        pltpu.VMEM_SHARED((n_nodes, width), dtype),
        pltpu.SemaphoreType.DMA((nbuf,)),
        pltpu.SemaphoreType.DMA((nbuf,)),
    ]
    if deg_width:
        out_type.append(
            jax.ShapeDtypeStruct((n_nodes, deg_width), jnp.float32))
        scratch += [
            pltpu.VMEM((k, deg_width), jnp.float32),
            pltpu.VMEM_SHARED((n_nodes, deg_width), jnp.float32),
            pltpu.SemaphoreType.DMA((nbuf,)),
        ]

    @functools.partial(pl.kernel, mesh=mesh, out_type=out_type,
                       compiler_params=pltpu.CompilerParams(
                           use_tc_tiling_on_sc=False,
                           skip_device_barrier=True),
                       scratch_types=scratch)
    def agg(*refs):
        if deg_width:
            (table_hbm, src_hbm, dst_hbm, zeros_hbm, zdeg_hbm, ones_hbm,
             out_hbm, deg_hbm, src_v, dst_v, rows_v, acc_sh, gsem, ssem,
             ones_v, deg_sh, dsem) = refs
        else:
            (table_hbm, src_hbm, dst_hbm, zeros_hbm, out_hbm,
             src_v, dst_v, rows_v, acc_sh, gsem, ssem) = refs
        c = lax.axis_index("c")
        s = lax.axis_index("s")
        tile_rows = pl.ds(s * rows_per_tile, rows_per_tile)
        pltpu.sync_copy(zeros_hbm, acc_sh.at[tile_rows])
        if deg_width:

            @pl.when(c == 0)
            def _():
                pltpu.sync_copy(zdeg_hbm, deg_sh.at[tile_rows])
                pltpu.sync_copy(ones_hbm, ones_v)

        plsc.subcore_barrier()

        for phase in range(n_phases):
            if split:
                base = s * n_chunks + phase * pchunks
                pltpu.sync_copy(src_hbm.at[pl.ds(base * k, pchunks * k)], src_v)

                def fix(i, carry):
                    v = src_v[pl.ds(i * 16, 16)]
                    src_v[pl.ds(i * 16, 16)] = v + v + c
                    return carry

                lax.fori_loop(0, pchunks * k // 16, fix, 0)
                sidx = lambda j: src_v.at[pl.ds(j * k, k)]
            else:
                base = (s * nc + c) * n_chunks + phase * pchunks
                pltpu.sync_copy(src_hbm.at[pl.ds(base, pchunks)], src_v)
                sidx = lambda j: src_v.at[j]
            pltpu.sync_copy(dst_hbm.at[pl.ds(base, pchunks)], dst_v)

            for b in range(nbuf):
                pltpu.async_copy(table_hbm.at[sidx(b)], rows_v.at[b],
                                 gsem.at[b])

            def body(g, carry):
                j0 = g * nbuf
                for b in range(nbuf):
                    j = j0 + b
                    pltpu.make_async_copy(
                        table_hbm.at[sidx(j)], rows_v.at[b], gsem.at[b]).wait()
                    pltpu.async_copy(
                        rows_v.at[b], acc_sh.at[dst_v.at[j]], ssem.at[b], add=True)
                    if deg_width:

                        @pl.when(c == 0)
                        def _():
                            pltpu.async_copy(
                                ones_v, deg_sh.at[dst_v.at[j]], dsem.at[b],
                                add=True)
                for b in range(nbuf):
                    j2 = j0 + nbuf + b

                    @pl.when(j2 < pchunks)
                    def _():
                        pltpu.make_async_copy(
                            rows_v.at[b], acc_sh.at[dst_v.at[j0 + b]],
                            ssem.at[b]).wait()
                        if deg_width:

                            @pl.when(c == 0)
                            def _():
                                pltpu.make_async_copy(
                                    ones_v, deg_sh.at[dst_v.at[j0 + b]],
                                    dsem.at[b]).wait()

                        pltpu.async_copy(
                            table_hbm.at[sidx(j2)], rows_v.at[b], gsem.at[b])

                return carry

            lax.fori_loop(0, pchunks // nbuf, body, 0)
            for b in range(nbuf):
                pltpu.make_async_copy(
                    rows_v.at[b], acc_sh.at[dst_v.at[pchunks - nbuf + b]],
                    ssem.at[b]).wait()
                if deg_width:

                    @pl.when(c == 0)
                    def _():
                        pltpu.make_async_copy(
                            ones_v, deg_sh.at[dst_v.at[pchunks - nbuf + b]],
                            dsem.at[b]).wait()

        plsc.subcore_barrier()
        pltpu.sync_copy(acc_sh.at[tile_rows], out_hbm.at[c, tile_rows])
        if deg_width:

            @pl.when(c == 0)
            def _():
                pltpu.sync_copy(deg_sh.at[tile_rows], deg_hbm.at[tile_rows])

    return agg


def _dense_mid(x, xagg, dego, ws1, wn1, b1, ws2, wn2, bn):
    n, d = x.shape
    h = ws1.shape[0]
    cdim = ws2.shape[0]
    hw = xagg.shape[-1]
    dw = dego.shape[-1]
    grid = n // bn

    def body(x_ref, agg_ref, deg_ref, ws1_ref, wn1_ref, b1_ref, ws2_ref,
             wn2_ref, h1_ref, h1r_ref, z2a_ref, hs2_ref):
        xagg_b = jnp.concatenate(
            [agg_ref[0].astype(jnp.float32), agg_ref[1].astype(jnp.float32)],
            axis=1)
        deg = jnp.maximum(deg_ref[:, 0:1], 1.0)
        mean = xagg_b / deg
        h1 = (lax.dot_general(x_ref[...], ws1_ref[...], _NT,
                              preferred_element_type=jnp.float32)
              + lax.dot_general(mean, wn1_ref[...], _NT,
                                preferred_element_type=jnp.float32)
              + b1_ref[...])
        h1_ref[...] = h1
        h1r = jnp.maximum(h1, 0.0)
        h1r_ref[...] = h1r
        z2 = lax.dot_general(h1r, wn2_ref[...], _NT,
                             preferred_element_type=jnp.float32)
        hs2_ref[...] = lax.dot_general(h1r, ws2_ref[...], _NT,
                                       preferred_element_type=jnp.float32)
        z2a_ref[...] = jnp.concatenate(
            [z2, jnp.ones((bn, cdim), jnp.float32)], axis=1)

    return pl.pallas_call(
        body,
        grid=(grid,),
        in_specs=[
            pl.BlockSpec((bn, d), lambda i: (i, 0)),
            pl.BlockSpec((2, bn, hw), lambda i: (0, i, 0)),
            pl.BlockSpec((bn, dw), lambda i: (i, 0)),
            pl.BlockSpec((h, d), lambda i: (0, 0)),
            pl.BlockSpec((h, d), lambda i: (0, 0)),
            pl.BlockSpec((1, h), lambda i: (0, 0)),
            pl.BlockSpec((cdim, h), lambda i: (0, 0)),
            pl.BlockSpec((cdim, h), lambda i: (0, 0)),
        ],
        out_specs=[
            pl.BlockSpec((bn, h), lambda i: (i, 0)),
            pl.BlockSpec((bn, h), lambda i: (i, 0)),
            pl.BlockSpec((bn, 2 * cdim), lambda i: (i, 0)),
            pl.BlockSpec((bn, cdim), lambda i: (i, 0)),
        ],
        out_shape=[
            jax.ShapeDtypeStruct((n, h), jnp.float32),
            jax.ShapeDtypeStruct((n, h), jnp.float32),
            jax.ShapeDtypeStruct((n, 2 * cdim), jnp.float32),
            jax.ShapeDtypeStruct((n, cdim), jnp.float32),
        ],
    )(x, xagg, dego, ws1, wn1, b1, ws2, wn2)


def _dense_final(hs2, agg2, b2, bn):
    n, cdim = hs2.shape
    wa = agg2.shape[-1]
    grid = n // bn

    def body(hs2_ref, agg_ref, b2_ref, h2_ref):
        a = agg_ref[0] + agg_ref[1]
        deg = jnp.maximum(a[:, cdim:cdim + 1], 1.0)
        h2_ref[...] = hs2_ref[...] + a[:, :cdim] / deg + b2_ref[...]

    return pl.pallas_call(
        body,
        grid=(grid,),
        in_specs=[
            pl.BlockSpec((bn, cdim), lambda i: (i, 0)),
            pl.BlockSpec((2, bn, wa), lambda i: (0, i, 0)),
            pl.BlockSpec((1, cdim), lambda i: (0, 0)),
        ],
        out_specs=pl.BlockSpec((bn, cdim), lambda i: (i, 0)),
        out_shape=jax.ShapeDtypeStruct((n, cdim), jnp.float32),
    )(hs2, agg2, b2)


def kernel(x, edge_index, W_self1, W_neigh1, b1, W_self2, W_neigh2, b2):
    n, d = x.shape
    e = edge_index.shape[1]
    h = W_self1.shape[0]
    cdim = W_self2.shape[0]
    info = plsc.get_sparse_core_info()
    nc, ns = info.num_cores, info.num_subcores

    hw = d // 2
    dw = 8
    tab = x.astype(jnp.bfloat16).reshape(nc * n, hw)

    k1, k2 = 200, 200
    src = edge_index[0]
    srcA = jnp.stack([2 * src, 2 * src + 1]).reshape(nc, e // k1, k1)
    dstA = edge_index[1].reshape(e // k1, k1)
    agg_fn1 = _make_sc_agg(n, e, hw, nc, ns, k=k1, nbuf=5, n_phases=1,
                           split=True, dtype=jnp.bfloat16, deg_width=dw)
    xagg, dego = agg_fn1(
        tab, src, dstA,
        jnp.zeros((n // ns, hw), jnp.bfloat16),
        jnp.zeros((n // ns, dw), jnp.float32),
        jnp.ones((k1, dw), jnp.float32))

    h1, h1r, z2a, hs2 = _dense_mid(
        x, xagg, dego, W_self1, W_neigh1, b1.reshape(1, h),
        W_self2, W_neigh2, bn=1000)

    w2 = 2 * cdim
    srcC = src.reshape(e // k2, k2)
    dstC = edge_index[1].reshape(e // k2, k2)
    agg_fn2 = _make_sc_agg(n, e, w2, nc, ns, k=k2, nbuf=5, n_phases=1,
                           split=False)
    (agg2,) = agg_fn2(z2a, srcC, dstC, jnp.zeros((n // ns, w2), jnp.float32))

    h2 = _dense_final(hs2, agg2, b2.reshape(1, cdim), bn=1000)
    return (h2, h1, h1r)


# speedup vs baseline: 1.3066x; 1.0277x over previous
import functools

import numpy as np

import jax
import jax.numpy as jnp
from jax import lax
from jax.experimental import pallas as pl
from jax.experimental.pallas import tpu as pltpu
from jax.experimental.pallas import tpu_sc as plsc

_NT = (((1,), (1,)), ((), ()))


def _make_sc_agg(n_nodes, n_edges, width, nc, ns, k, nbuf, n_phases, split,
                 dtype=jnp.float32, deg_width=0):
    edges_per_worker = n_edges // (ns if split else nc * ns)
    n_chunks = edges_per_worker // k
    assert edges_per_worker % k == 0 and n_chunks % n_phases == 0
    pchunks = n_chunks // n_phases
    assert pchunks % nbuf == 0
    rows_per_tile = n_nodes // ns
    mesh = plsc.VectorSubcoreMesh(core_axis_name="c", subcore_axis_name="s")

    out_type = [jax.ShapeDtypeStruct((nc, n_nodes, width), dtype)]
    scratch = [
        pltpu.VMEM((pchunks, k), jnp.int32),
        pltpu.VMEM((pchunks, k), jnp.int32),
        pltpu.VMEM((nbuf, k, width), dtype),
        pltpu.VMEM_SHARED((n_nodes, width), dtype),
        pltpu.SemaphoreType.DMA((nbuf,)),
        pltpu.SemaphoreType.DMA((nbuf,)),
    ]
    if deg_width:
        out_type.append(
            jax.ShapeDtypeStruct((n_nodes, deg_width), jnp.float32))
        scratch += [
            pltpu.VMEM((k, deg_width), jnp.float32),
            pltpu.VMEM_SHARED((n_nodes, deg_width), jnp.float32),
            pltpu.SemaphoreType.DMA((nbuf,)),
        ]

    @functools.partial(pl.kernel, mesh=mesh, out_type=out_type,
                       compiler_params=pltpu.CompilerParams(
                           use_tc_tiling_on_sc=False,
                           skip_device_barrier=True),
                       scratch_types=scratch)
    def agg(*refs):
        if deg_width:
            (table_hbm, ei_hbm, zeros_hbm, zdeg_hbm, ones_hbm,
             out_hbm, deg_hbm, src_v, dst_v, rows_v, acc_sh, gsem, ssem,
             ones_v, deg_sh, dsem) = refs
        else:
            (table_hbm, ei_hbm, zeros_hbm, out_hbm,
             src_v, dst_v, rows_v, acc_sh, gsem, ssem) = refs
        c = lax.axis_index("c")
        s = lax.axis_index("s")
        tile_rows = pl.ds(s * rows_per_tile, rows_per_tile)
        pltpu.sync_copy(zeros_hbm, acc_sh.at[tile_rows])
        if deg_width:

            @pl.when(c == 0)
            def _():
                pltpu.sync_copy(zdeg_hbm, deg_sh.at[tile_rows])
                pltpu.sync_copy(ones_hbm, ones_v)

        plsc.subcore_barrier()

        for phase in range(n_phases):
            if split:
                base = s * n_chunks + phase * pchunks
                pltpu.sync_copy(ei_hbm.at[0, pl.ds(base, pchunks)], src_v)
                pltpu.sync_copy(ei_hbm.at[1, pl.ds(base, pchunks)], dst_v)
                kq = k // 16

                def fix(i, carry):
                    r = lax.div(i, kq)
                    q = lax.rem(i, kq)
                    v = src_v[r, pl.ds(q * 16, 16)]
                    src_v[r, pl.ds(q * 16, 16)] = v + v + c
                    return carry

                lax.fori_loop(0, pchunks * kq, fix, 0)
                sidx = lambda j: src_v.at[j]
            else:
                base = (s * nc + c) * n_chunks + phase * pchunks
                pltpu.sync_copy(ei_hbm.at[0, pl.ds(base, pchunks)], src_v)
                pltpu.sync_copy(ei_hbm.at[1, pl.ds(base, pchunks)], dst_v)
                sidx = lambda j: src_v.at[j]

            for b in range(nbuf):
                pltpu.async_copy(table_hbm.at[sidx(b)], rows_v.at[b],
                                 gsem.at[b])

            def body(g, carry):
                j0 = g * nbuf
                for b in range(nbuf):
                    j = j0 + b
                    pltpu.make_async_copy(
                        table_hbm.at[sidx(j)], rows_v.at[b], gsem.at[b]).wait()
                    pltpu.async_copy(
                        rows_v.at[b], acc_sh.at[dst_v.at[j]], ssem.at[b], add=True)
                    if deg_width:

                        @pl.when(c == 0)
                        def _():
                            pltpu.async_copy(
                                ones_v, deg_sh.at[dst_v.at[j]], dsem.at[b],
                                add=True)
                for b in range(nbuf):
                    j2 = j0 + nbuf + b

                    @pl.when(j2 < pchunks)
                    def _():
                        pltpu.make_async_copy(
                            rows_v.at[b], acc_sh.at[dst_v.at[j0 + b]],
                            ssem.at[b]).wait()
                        if deg_width:

                            @pl.when(c == 0)
                            def _():
                                pltpu.make_async_copy(
                                    ones_v, deg_sh.at[dst_v.at[j0 + b]],
                                    dsem.at[b]).wait()

                        pltpu.async_copy(
                            table_hbm.at[sidx(j2)], rows_v.at[b], gsem.at[b])

                return carry

            lax.fori_loop(0, pchunks // nbuf, body, 0)
            for b in range(nbuf):
                pltpu.make_async_copy(
                    rows_v.at[b], acc_sh.at[dst_v.at[pchunks - nbuf + b]],
                    ssem.at[b]).wait()
                if deg_width:

                    @pl.when(c == 0)
                    def _():
                        pltpu.make_async_copy(
                            ones_v, deg_sh.at[dst_v.at[pchunks - nbuf + b]],
                            dsem.at[b]).wait()

        plsc.subcore_barrier()
        pltpu.sync_copy(acc_sh.at[tile_rows], out_hbm.at[c, tile_rows])
        if deg_width:

            @pl.when(c == 0)
            def _():
                pltpu.sync_copy(deg_sh.at[tile_rows], deg_hbm.at[tile_rows])

    return agg


def _dense_mid(x, xagg, dego, ws1, wn1, b1, ws2, wn2, bn):
    n, d = x.shape
    h = ws1.shape[0]
    cdim = ws2.shape[0]
    hw = xagg.shape[-1]
    dw = dego.shape[-1]
    grid = n // bn

    def body(x_ref, agg_ref, deg_ref, ws1_ref, wn1_ref, b1_ref, ws2_ref,
             wn2_ref, h1_ref, h1r_ref, z2a_ref, hs2_ref):
        xagg_b = jnp.concatenate(
            [agg_ref[0].astype(jnp.float32), agg_ref[1].astype(jnp.float32)],
            axis=1)
        deg = jnp.maximum(deg_ref[:, 0:1], 1.0)
        mean = xagg_b / deg
        h1 = (lax.dot_general(x_ref[...], ws1_ref[...], _NT,
                              preferred_element_type=jnp.float32)
              + lax.dot_general(mean, wn1_ref[...], _NT,
                                preferred_element_type=jnp.float32)
              + b1_ref[...])
        h1_ref[...] = h1
        h1r = jnp.maximum(h1, 0.0)
        h1r_ref[...] = h1r
        z2 = lax.dot_general(h1r, wn2_ref[...], _NT,
                             preferred_element_type=jnp.float32)
        hs2_ref[...] = lax.dot_general(h1r, ws2_ref[...], _NT,
                                       preferred_element_type=jnp.float32)
        z2a_ref[...] = jnp.concatenate(
            [z2, jnp.ones((bn, cdim), jnp.float32)], axis=1)

    return pl.pallas_call(
        body,
        grid=(grid,),
        in_specs=[
            pl.BlockSpec((bn, d), lambda i: (i, 0)),
            pl.BlockSpec((2, bn, hw), lambda i: (0, i, 0)),
            pl.BlockSpec((bn, dw), lambda i: (i, 0)),
            pl.BlockSpec((h, d), lambda i: (0, 0)),
            pl.BlockSpec((h, d), lambda i: (0, 0)),
            pl.BlockSpec((1, h), lambda i: (0, 0)),
            pl.BlockSpec((cdim, h), lambda i: (0, 0)),
            pl.BlockSpec((cdim, h), lambda i: (0, 0)),
        ],
        out_specs=[
            pl.BlockSpec((bn, h), lambda i: (i, 0)),
            pl.BlockSpec((bn, h), lambda i: (i, 0)),
            pl.BlockSpec((bn, 2 * cdim), lambda i: (i, 0)),
            pl.BlockSpec((bn, cdim), lambda i: (i, 0)),
        ],
        out_shape=[
            jax.ShapeDtypeStruct((n, h), jnp.float32),
            jax.ShapeDtypeStruct((n, h), jnp.float32),
            jax.ShapeDtypeStruct((n, 2 * cdim), jnp.float32),
            jax.ShapeDtypeStruct((n, cdim), jnp.float32),
        ],
    )(x, xagg, dego, ws1, wn1, b1, ws2, wn2)


def _dense_final(hs2, agg2, b2, bn):
    n, cdim = hs2.shape
    wa = agg2.shape[-1]
    grid = n // bn

    def body(hs2_ref, agg_ref, b2_ref, h2_ref):
        a = agg_ref[0] + agg_ref[1]
        deg = jnp.maximum(a[:, cdim:cdim + 1], 1.0)
        h2_ref[...] = hs2_ref[...] + a[:, :cdim] / deg + b2_ref[...]

    return pl.pallas_call(
        body,
        grid=(grid,),
        in_specs=[
            pl.BlockSpec((bn, cdim), lambda i: (i, 0)),
            pl.BlockSpec((2, bn, wa), lambda i: (0, i, 0)),
            pl.BlockSpec((1, cdim), lambda i: (0, 0)),
        ],
        out_specs=pl.BlockSpec((bn, cdim), lambda i: (i, 0)),
        out_shape=jax.ShapeDtypeStruct((n, cdim), jnp.float32),
    )(hs2, agg2, b2)


def kernel(x, edge_index, W_self1, W_neigh1, b1, W_self2, W_neigh2, b2):
    n, d = x.shape
    e = edge_index.shape[1]
    h = W_self1.shape[0]
    cdim = W_self2.shape[0]
    info = plsc.get_sparse_core_info()
    nc, ns = info.num_cores, info.num_subcores

    hw = d // 2
    dw = 8
    tab = x.astype(jnp.bfloat16).reshape(nc * n, hw)

    k1, k2 = 160, 200
    ei3a = edge_index.reshape(2, e // k1, k1)
    ei3c = edge_index.reshape(2, e // k2, k2)
    agg_fn1 = _make_sc_agg(n, e, hw, nc, ns, k=k1, nbuf=5, n_phases=1,
                           split=True, dtype=jnp.bfloat16, deg_width=dw)
    xagg, dego = agg_fn1(
        tab, ei3a,
        jnp.zeros((n // ns, hw), jnp.bfloat16),
        jnp.zeros((n // ns, dw), jnp.float32),
        jnp.ones((k1, dw), jnp.float32))

    h1, h1r, z2a, hs2 = _dense_mid(
        x, xagg, dego, W_self1, W_neigh1, b1.reshape(1, h),
        W_self2, W_neigh2, bn=1000)

    w2 = 2 * cdim
    agg_fn2 = _make_sc_agg(n, e, w2, nc, ns, k=k2, nbuf=5, n_phases=1,
                           split=False)
    (agg2,) = agg_fn2(z2a, ei3c, jnp.zeros((n // ns, w2), jnp.float32))

    h2 = _dense_final(hs2, agg2, b2.reshape(1, cdim), bn=1000)
    return (h2, h1, h1r)


# speedup vs baseline: 1.3339x; 1.0209x over previous
import functools

import numpy as np

import jax
import jax.numpy as jnp
from jax import lax
from jax.experimental import pallas as pl
from jax.experimental.pallas import tpu as pltpu
from jax.experimental.pallas import tpu_sc as plsc

_NT = (((1,), (1,)), ((), ()))


def _make_sc_agg(n_nodes, n_edges, width, nc, ns, k, nbuf, n_phases, split,
                 dtype=jnp.float32, deg_width=0):
    edges_per_worker = n_edges // (ns if split else nc * ns)
    n_chunks = edges_per_worker // k
    assert edges_per_worker % k == 0 and n_chunks % n_phases == 0
    pchunks = n_chunks // n_phases
    assert pchunks % nbuf == 0
    rows_per_tile = n_nodes // ns
    mesh = plsc.VectorSubcoreMesh(core_axis_name="c", subcore_axis_name="s")

    out_type = [jax.ShapeDtypeStruct((nc, n_nodes, width), dtype)]
    scratch = [
        pltpu.VMEM((pchunks, k), jnp.int32),
        pltpu.VMEM((pchunks, k), jnp.int32),
        pltpu.VMEM((nbuf, k, width), dtype),
        pltpu.VMEM_SHARED((n_nodes, width), dtype),
        pltpu.SemaphoreType.DMA((nbuf,)),
        pltpu.SemaphoreType.DMA((nbuf,)),
    ]
    if deg_width:
        out_type.append(
            jax.ShapeDtypeStruct((n_nodes, deg_width), jnp.float32))
        scratch += [
            pltpu.VMEM((k, deg_width), jnp.float32),
            pltpu.VMEM_SHARED((n_nodes, deg_width), jnp.float32),
            pltpu.SemaphoreType.DMA((nbuf,)),
        ]

    @functools.partial(pl.kernel, mesh=mesh, out_type=out_type,
                       compiler_params=pltpu.CompilerParams(
                           use_tc_tiling_on_sc=False,
                           skip_device_barrier=True),
                       scratch_types=scratch)
    def agg(*refs):
        if deg_width:
            (table_hbm, ei_hbm, zeros_hbm, zdeg_hbm, ones_hbm,
             out_hbm, deg_hbm, src_v, dst_v, rows_v, acc_sh, gsem, ssem,
             ones_v, deg_sh, dsem) = refs
        else:
            (table_hbm, ei_hbm, zeros_hbm, out_hbm,
             src_v, dst_v, rows_v, acc_sh, gsem, ssem) = refs
        c = lax.axis_index("c")
        s = lax.axis_index("s")
        tile_rows = pl.ds(s * rows_per_tile, rows_per_tile)
        pltpu.sync_copy(zeros_hbm, acc_sh.at[tile_rows])
        if deg_width:

            @pl.when(c == 0)
            def _():
                pltpu.sync_copy(zdeg_hbm, deg_sh.at[tile_rows])
                pltpu.sync_copy(ones_hbm, ones_v)

        plsc.subcore_barrier()

        for phase in range(n_phases):
            if split:
                base = s * n_chunks + phase * pchunks
                pltpu.sync_copy(ei_hbm.at[0, pl.ds(base, pchunks)], src_v)
                pltpu.sync_copy(ei_hbm.at[1, pl.ds(base, pchunks)], dst_v)
                kq = k // 16

                def fix(i, carry):
                    r = lax.div(i, kq)
                    q = lax.rem(i, kq)
                    v = src_v[r, pl.ds(q * 16, 16)]
                    src_v[r, pl.ds(q * 16, 16)] = v + v + c
                    return carry

                lax.fori_loop(0, pchunks * kq, fix, 0)
                sidx = lambda j: src_v.at[j]
            else:
                base = (s * nc + c) * n_chunks + phase * pchunks
                pltpu.sync_copy(ei_hbm.at[0, pl.ds(base, pchunks)], src_v)
                pltpu.sync_copy(ei_hbm.at[1, pl.ds(base, pchunks)], dst_v)
                sidx = lambda j: src_v.at[j]

            for b in range(nbuf):
                pltpu.async_copy(table_hbm.at[sidx(b)], rows_v.at[b],
                                 gsem.at[b])

            def body(g, carry):
                j0 = g * nbuf
                for b in range(nbuf):
                    j = j0 + b
                    pltpu.make_async_copy(
                        table_hbm.at[sidx(j)], rows_v.at[b], gsem.at[b]).wait()
                    pltpu.async_copy(
                        rows_v.at[b], acc_sh.at[dst_v.at[j]], ssem.at[b], add=True)
                    if deg_width:

                        @pl.when(c == 0)
                        def _():
                            pltpu.async_copy(
                                ones_v, deg_sh.at[dst_v.at[j]], dsem.at[b],
                                add=True)
                for b in range(nbuf):
                    j2 = j0 + nbuf + b

                    @pl.when(j2 < pchunks)
                    def _():
                        pltpu.make_async_copy(
                            rows_v.at[b], acc_sh.at[dst_v.at[j0 + b]],
                            ssem.at[b]).wait()
                        if deg_width:

                            @pl.when(c == 0)
                            def _():
                                pltpu.make_async_copy(
                                    ones_v, deg_sh.at[dst_v.at[j0 + b]],
                                    dsem.at[b]).wait()

                        pltpu.async_copy(
                            table_hbm.at[sidx(j2)], rows_v.at[b], gsem.at[b])

                return carry

            lax.fori_loop(0, pchunks // nbuf, body, 0)
            for b in range(nbuf):
                pltpu.make_async_copy(
                    rows_v.at[b], acc_sh.at[dst_v.at[pchunks - nbuf + b]],
                    ssem.at[b]).wait()
                if deg_width:

                    @pl.when(c == 0)
                    def _():
                        pltpu.make_async_copy(
                            ones_v, deg_sh.at[dst_v.at[pchunks - nbuf + b]],
                            dsem.at[b]).wait()

        plsc.subcore_barrier()
        pltpu.sync_copy(acc_sh.at[tile_rows], out_hbm.at[c, tile_rows])
        if deg_width:

            @pl.when(c == 0)
            def _():
                pltpu.sync_copy(deg_sh.at[tile_rows], deg_hbm.at[tile_rows])

    return agg


def _dense_mid(x, xagg, dego, ws1, wn1, b1, ws2, wn2, bn):
    n, d = x.shape
    h = ws1.shape[0]
    cdim = ws2.shape[0]
    hw = xagg.shape[-1]
    dw = dego.shape[-1]
    grid = n // bn

    def body(x_ref, agg_ref, deg_ref, ws1_ref, wn1_ref, b1_ref, ws2_ref,
             wn2_ref, h1_ref, h1r_ref, z2a_ref, hs2_ref):
        xagg_b = jnp.concatenate(
            [agg_ref[0].astype(jnp.float32), agg_ref[1].astype(jnp.float32)],
            axis=1)
        deg = jnp.maximum(deg_ref[:, 0:1], 1.0)
        mean = xagg_b / deg
        h1 = (lax.dot_general(x_ref[...], ws1_ref[...], _NT,
                              preferred_element_type=jnp.float32)
              + lax.dot_general(mean, wn1_ref[...], _NT,
                                preferred_element_type=jnp.float32)
              + b1_ref[...])
        h1_ref[...] = h1
        h1r = jnp.maximum(h1, 0.0)
        h1r_ref[...] = h1r
        z2 = lax.dot_general(h1r, wn2_ref[...], _NT,
                             preferred_element_type=jnp.float32)
        hs2_ref[...] = lax.dot_general(h1r, ws2_ref[...], _NT,
                                       preferred_element_type=jnp.float32)
        z2a_ref[...] = jnp.concatenate(
            [z2, jnp.ones((bn, cdim), jnp.float32)], axis=1)

    return pl.pallas_call(
        body,
        grid=(grid,),
        in_specs=[
            pl.BlockSpec((bn, d), lambda i: (i, 0)),
            pl.BlockSpec((2, bn, hw), lambda i: (0, i, 0)),
            pl.BlockSpec((bn, dw), lambda i: (i, 0)),
            pl.BlockSpec((h, d), lambda i: (0, 0)),
            pl.BlockSpec((h, d), lambda i: (0, 0)),
            pl.BlockSpec((1, h), lambda i: (0, 0)),
            pl.BlockSpec((cdim, h), lambda i: (0, 0)),
            pl.BlockSpec((cdim, h), lambda i: (0, 0)),
        ],
        out_specs=[
            pl.BlockSpec((bn, h), lambda i: (i, 0)),
            pl.BlockSpec((bn, h), lambda i: (i, 0)),
            pl.BlockSpec((bn, 2 * cdim), lambda i: (i, 0)),
            pl.BlockSpec((bn, cdim), lambda i: (i, 0)),
        ],
        out_shape=[
            jax.ShapeDtypeStruct((n, h), jnp.float32),
            jax.ShapeDtypeStruct((n, h), jnp.float32),
            jax.ShapeDtypeStruct((n, 2 * cdim), jnp.float32),
            jax.ShapeDtypeStruct((n, cdim), jnp.float32),
        ],
    )(x, xagg, dego, ws1, wn1, b1, ws2, wn2)


def _dense_final(hs2, agg2, b2, bn):
    n, cdim = hs2.shape
    wa = agg2.shape[-1]
    grid = n // bn

    def body(hs2_ref, agg_ref, b2_ref, h2_ref):
        a = agg_ref[0] + agg_ref[1]
        deg = jnp.maximum(a[:, cdim:cdim + 1], 1.0)
        h2_ref[...] = hs2_ref[...] + a[:, :cdim] / deg + b2_ref[...]

    return pl.pallas_call(
        body,
        grid=(grid,),
        in_specs=[
            pl.BlockSpec((bn, cdim), lambda i: (i, 0)),
            pl.BlockSpec((2, bn, wa), lambda i: (0, i, 0)),
            pl.BlockSpec((1, cdim), lambda i: (0, 0)),
        ],
        out_specs=pl.BlockSpec((bn, cdim), lambda i: (i, 0)),
        out_shape=jax.ShapeDtypeStruct((n, cdim), jnp.float32),
    )(hs2, agg2, b2)


def kernel(x, edge_index, W_self1, W_neigh1, b1, W_self2, W_neigh2, b2):
    n, d = x.shape
    e = edge_index.shape[1]
    h = W_self1.shape[0]
    cdim = W_self2.shape[0]
    info = plsc.get_sparse_core_info()
    nc, ns = info.num_cores, info.num_subcores

    hw = d // 2
    dw = 8
    tab = x.astype(jnp.bfloat16).reshape(nc * n, hw)

    k1, k2 = 160, 200
    ei3a = edge_index.reshape(2, e // k1, k1)
    ei3c = edge_index.reshape(2, e // k2, k2)
    agg_fn1 = _make_sc_agg(n, e, hw, nc, ns, k=k1, nbuf=5, n_phases=1,
                           split=True, dtype=jnp.bfloat16, deg_width=dw)
    xagg, dego = agg_fn1(
        tab, ei3a,
        jnp.zeros((n // ns, hw), jnp.bfloat16),
        jnp.zeros((n // ns, dw), jnp.float32),
        jnp.ones((k1, dw), jnp.float32))

    h1, h1r, z2a, hs2 = _dense_mid(
        x, xagg, dego, W_self1, W_neigh1, b1.reshape(1, h),
        W_self2, W_neigh2, bn=2000)

    w2 = 2 * cdim
    agg_fn2 = _make_sc_agg(n, e, w2, nc, ns, k=k2, nbuf=5, n_phases=1,
                           split=False)
    (agg2,) = agg_fn2(z2a, ei3c, jnp.zeros((n // ns, w2), jnp.float32))

    h2 = _dense_final(hs2, agg2, b2.reshape(1, cdim), bn=2000)
    return (h2, h1, h1r)
